# baseline (speedup 1.0000x reference)
import jax
import jax.numpy as jnp
from jax.experimental import pallas as pl
from jax.experimental.pallas import tpu as pltpu

_ALPHA = 0.2
_R = 512
_INTERPRET = False


def _body(adj_ref, x_ref, Wcat_ref, A1_ref, A2_ref, Wout_ref,
          a1o_ref, a2o_ref, out_ref,
          h_scr, haug_scr, vT_scr, qT_scr, h1_scr,
          h2_scr, haug2_scr, v2T_scr, q2T_scr):
    b = pl.program_id(0)
    n = x_ref.shape[0]
    nheads = A1_ref.shape[1]
    nhid = A1_ref.shape[0] // nheads
    nclass = Wout_ref.shape[1]
    nb = n // _R

    @pl.when(b == 0)
    def _prologue1():
        h = jnp.dot(x_ref[...], Wcat_ref[...],
                    preferred_element_type=jnp.float32)
        h_scr[...] = h
        haug_scr[...] = jnp.ones(haug_scr.shape, dtype=jnp.bfloat16)
        for k in range(nheads):
            haug_scr[:, k * (nhid + 1):k * (nhid + 1) + nhid] = (
                h[:, k * nhid:(k + 1) * nhid].astype(jnp.bfloat16))
        t = jnp.dot(h, A2_ref[...],
                    preferred_element_type=jnp.float32)
        tT = jnp.transpose(t)
        vT_scr[...] = jnp.exp(-tT).astype(jnp.bfloat16)
        qT_scr[...] = jnp.exp(-_ALPHA * tT).astype(jnp.bfloat16)

    @pl.when(b < nb)
    def _layer1():
        adjf = (adj_ref[...] != 0).astype(jnp.bfloat16)
        h_blk = h_scr[pl.ds(b * _R, _R), :]
        s_blk = jnp.dot(h_blk, A1_ref[...],
                        preferred_element_type=jnp.float32)
        r_blk = jnp.exp((1.0 - _ALPHA) * s_blk).astype(jnp.bfloat16)
        for k in range(nheads):
            E = (jnp.minimum(vT_scr[k:k + 1, :],
                             r_blk[:, k:k + 1] * qT_scr[k:k + 1, :])
                 * adjf)
            hk = haug_scr[:, k * (nhid + 1):(k + 1) * (nhid + 1)]
            acc = jnp.dot(E, hk, preferred_element_type=jnp.float32)
            hp = acc[:, :nhid] / acc[:, nhid:nhid + 1]
            h1_scr[pl.ds(b * _R, _R), k * nhid:(k + 1) * nhid] = (
                jnp.where(hp > 0, hp, jnp.exp(hp) - 1.0))

    @pl.when(b == nb)
    def _prologue2():
        h2 = jnp.dot(h1_scr[...], Wout_ref[...],
                     preferred_element_type=jnp.float32)
        h2_scr[...] = h2
        ones = jnp.ones((n, 1), dtype=jnp.float32)
        haug2_scr[...] = jnp.concatenate([h2, ones],
                                         axis=1).astype(jnp.bfloat16)
        t2 = jnp.dot(h2, a2o_ref[...],
                     preferred_element_type=jnp.float32)
        t2T = jnp.transpose(t2)
        v2T_scr[...] = jnp.exp(-t2T).astype(jnp.bfloat16)
        q2T_scr[...] = jnp.exp(-_ALPHA * t2T).astype(jnp.bfloat16)

    @pl.when(b >= nb)
    def _layer2():
        r = b - nb
        adjf = (adj_ref[...] != 0).astype(jnp.bfloat16)
        h2_blk = h2_scr[pl.ds(r * _R, _R), :]
        s2 = jnp.dot(h2_blk, a1o_ref[...],
                     preferred_element_type=jnp.float32)
        r2 = jnp.exp((1.0 - _ALPHA) * s2).astype(jnp.bfloat16)
        E = (jnp.minimum(v2T_scr[...], r2 * q2T_scr[...])
             * adjf)
        acc = jnp.dot(E, haug2_scr[...], preferred_element_type=jnp.float32)
        m = acc[:, :nclass] / acc[:, nclass:nclass + 1]
        e = jnp.where(m > 0, m, jnp.exp(m) - 1.0)
        lse = jnp.log(jnp.sum(jnp.exp(e), axis=1, keepdims=True))
        out_ref[...] = e - lse


def kernel(inputs, adj, W_heads, a_heads, W_out, a_out):
    x = inputs
    n, nfeat = x.shape
    nheads, _, nhid = W_heads.shape
    nclass = W_out.shape[1]

    Wcat = jnp.transpose(W_heads, (1, 0, 2)).reshape(nfeat, nheads * nhid)
    a1 = a_heads[:, 0, :nhid]
    a2 = a_heads[:, 0, nhid:]
    eye = jnp.eye(nheads, dtype=x.dtype)
    A1 = (eye[:, None, :] * a1[:, :, None]).reshape(nheads * nhid, nheads)
    A2 = (eye[:, None, :] * a2[:, :, None]).reshape(nheads * nhid, nheads)
    a1o = a_out[0, :nclass][:, None]
    a2o = a_out[0, nclass:][:, None]

    nb = n // _R
    full = lambda shape: pl.BlockSpec(shape, lambda b: (0, 0))
    logits = pl.pallas_call(
        _body,
        grid=(2 * nb,),
        in_specs=[
            pl.BlockSpec((_R, n), lambda b: (jax.lax.rem(b, nb), 0)),
            full((n, nfeat)),
            full((nfeat, nheads * nhid)),
            full((nheads * nhid, nheads)),
            full((nheads * nhid, nheads)),
            full((nheads * nhid, nclass)),
            full((nclass, 1)),
            full((nclass, 1)),
        ],
        out_specs=pl.BlockSpec((_R, nclass),
                               lambda b: (jax.lax.max(b - nb, 0), 0)),
        out_shape=jax.ShapeDtypeStruct((n, nclass), jnp.float32),
        scratch_shapes=[
            pltpu.VMEM((n, nheads * nhid), jnp.float32),
            pltpu.VMEM((n, nheads * (nhid + 1)), jnp.bfloat16),
            pltpu.VMEM((nheads, n), jnp.bfloat16),
            pltpu.VMEM((nheads, n), jnp.bfloat16),
            pltpu.VMEM((n, nheads * nhid), jnp.float32),
            pltpu.VMEM((n, nclass), jnp.float32),
            pltpu.VMEM((n, nclass + 1), jnp.bfloat16),
            pltpu.VMEM((1, n), jnp.bfloat16),
            pltpu.VMEM((1, n), jnp.bfloat16),
        ],
        interpret=_INTERPRET,
    )(adj, x, Wcat, A1, A2, W_out, a1o, a2o)
    return (logits, inputs)


# speedup vs baseline: 1.0387x; 1.0387x over previous
import jax
import jax.numpy as jnp
from jax.experimental import pallas as pl
from jax.experimental.pallas import tpu as pltpu

_ALPHA = 0.2
_R = 512
_INTERPRET = False


def _body(adj_ref, x_ref, Wcat_ref, A1_ref, A2_ref, Wout_ref,
          a1o_ref, a2o_ref, out_ref,
          h_scr, haug_scr, vT_scr, qT_scr, adjf_scr, h1_scr,
          h2_scr, haug2_scr, v2T_scr, q2T_scr):
    b = pl.program_id(0)
    n = x_ref.shape[0]
    nheads = A1_ref.shape[1]
    nhid = A1_ref.shape[0] // nheads
    nclass = Wout_ref.shape[1]
    nb = n // _R

    @pl.when(b == 0)
    def _prologue1():
        h = jnp.dot(x_ref[...], Wcat_ref[...],
                    preferred_element_type=jnp.float32)
        h_scr[...] = h
        haug_scr[...] = jnp.ones(haug_scr.shape, dtype=jnp.bfloat16)
        for k in range(nheads):
            haug_scr[:, k * (nhid + 1):k * (nhid + 1) + nhid] = (
                h[:, k * nhid:(k + 1) * nhid].astype(jnp.bfloat16))
        t = jnp.dot(h, A2_ref[...],
                    preferred_element_type=jnp.float32)
        tT = jnp.transpose(t)
        vT_scr[...] = jnp.exp(-tT).astype(jnp.bfloat16)
        qT_scr[...] = jnp.exp(-_ALPHA * tT).astype(jnp.bfloat16)

    @pl.when(b < nb)
    def _layer1():
        adjf = (adj_ref[...] != 0).astype(jnp.bfloat16)
        adjf_scr[pl.ds(b * _R, _R), :] = adjf
        h_blk = h_scr[pl.ds(b * _R, _R), :]
        s_blk = jnp.dot(h_blk, A1_ref[...],
                        preferred_element_type=jnp.float32)
        r_blk = jnp.exp((1.0 - _ALPHA) * s_blk).astype(jnp.bfloat16)
        for k in range(nheads):
            E = (jnp.minimum(vT_scr[k:k + 1, :],
                             r_blk[:, k:k + 1] * qT_scr[k:k + 1, :])
                 * adjf)
            hk = haug_scr[:, k * (nhid + 1):(k + 1) * (nhid + 1)]
            acc = jnp.dot(E, hk, preferred_element_type=jnp.float32)
            hp = acc[:, :nhid] / acc[:, nhid:nhid + 1]
            h1_scr[pl.ds(b * _R, _R), k * nhid:(k + 1) * nhid] = (
                jnp.where(hp > 0, hp, jnp.exp(hp) - 1.0))

    @pl.when(b == nb)
    def _prologue2():
        h2 = jnp.dot(h1_scr[...], Wout_ref[...],
                     preferred_element_type=jnp.float32)
        h2_scr[...] = h2
        ones = jnp.ones((n, 1), dtype=jnp.float32)
        haug2_scr[...] = jnp.concatenate([h2, ones],
                                         axis=1).astype(jnp.bfloat16)
        t2 = jnp.dot(h2, a2o_ref[...],
                     preferred_element_type=jnp.float32)
        t2T = jnp.transpose(t2)
        v2T_scr[...] = jnp.exp(-t2T).astype(jnp.bfloat16)
        q2T_scr[...] = jnp.exp(-_ALPHA * t2T).astype(jnp.bfloat16)

    @pl.when(b >= nb)
    def _layer2():
        r = b - nb
        adjf = adjf_scr[pl.ds(r * _R, _R), :]
        h2_blk = h2_scr[pl.ds(r * _R, _R), :]
        s2 = jnp.dot(h2_blk, a1o_ref[...],
                     preferred_element_type=jnp.float32)
        r2 = jnp.exp((1.0 - _ALPHA) * s2).astype(jnp.bfloat16)
        E = (jnp.minimum(v2T_scr[...], r2 * q2T_scr[...])
             * adjf)
        acc = jnp.dot(E, haug2_scr[...], preferred_element_type=jnp.float32)
        m = acc[:, :nclass] / acc[:, nclass:nclass + 1]
        e = jnp.where(m > 0, m, jnp.exp(m) - 1.0)
        lse = jnp.log(jnp.sum(jnp.exp(e), axis=1, keepdims=True))
        out_ref[...] = e - lse


def kernel(inputs, adj, W_heads, a_heads, W_out, a_out):
    x = inputs
    n, nfeat = x.shape
    nheads, _, nhid = W_heads.shape
    nclass = W_out.shape[1]

    Wcat = jnp.transpose(W_heads, (1, 0, 2)).reshape(nfeat, nheads * nhid)
    a1 = a_heads[:, 0, :nhid]
    a2 = a_heads[:, 0, nhid:]
    eye = jnp.eye(nheads, dtype=x.dtype)
    A1 = (eye[:, None, :] * a1[:, :, None]).reshape(nheads * nhid, nheads)
    A2 = (eye[:, None, :] * a2[:, :, None]).reshape(nheads * nhid, nheads)
    a1o = a_out[0, :nclass][:, None]
    a2o = a_out[0, nclass:][:, None]

    nb = n // _R
    full = lambda shape: pl.BlockSpec(shape, lambda b: (0, 0))
    logits = pl.pallas_call(
        _body,
        grid=(2 * nb,),
        in_specs=[
            pl.BlockSpec((_R, n), lambda b: (jax.lax.min(b, nb - 1), 0)),
            full((n, nfeat)),
            full((nfeat, nheads * nhid)),
            full((nheads * nhid, nheads)),
            full((nheads * nhid, nheads)),
            full((nheads * nhid, nclass)),
            full((nclass, 1)),
            full((nclass, 1)),
        ],
        out_specs=pl.BlockSpec((_R, nclass),
                               lambda b: (jax.lax.max(b - nb, 0), 0)),
        out_shape=jax.ShapeDtypeStruct((n, nclass), jnp.float32),
        scratch_shapes=[
            pltpu.VMEM((n, nheads * nhid), jnp.float32),
            pltpu.VMEM((n, nheads * (nhid + 1)), jnp.bfloat16),
            pltpu.VMEM((nheads, n), jnp.bfloat16),
            pltpu.VMEM((nheads, n), jnp.bfloat16),
            pltpu.VMEM((n, n), jnp.bfloat16),
            pltpu.VMEM((n, nheads * nhid), jnp.float32),
            pltpu.VMEM((n, nclass), jnp.float32),
            pltpu.VMEM((n, nclass + 1), jnp.bfloat16),
            pltpu.VMEM((1, n), jnp.bfloat16),
            pltpu.VMEM((1, n), jnp.bfloat16),
        ],
        interpret=_INTERPRET,
    )(adj, x, Wcat, A1, A2, W_out, a1o, a2o)
    return (logits, inputs)
